# baseline (speedup 1.0000x reference)
import functools

import jax
import jax.numpy as jnp
from jax import lax
from jax.experimental import pallas as pl
from jax.experimental.pallas import tpu as pltpu
from jax.experimental.pallas import tpu_sc as plsc

N = 10000
E = 320000
D_STATE = 128
D_MSG = 64
H = 128
G3 = 3 * H
DE = 144
NC, NS, L = 2, 16, 16
NW = NC * NS
EPT = E // NW
EB = 80
NB = EPT // EB
ROWS_PT = 624
ROWS_REM = N - NS * ROWS_PT

_INTERPRET = False


def _encode_body(state_ref, msg_ref, w1_ref, b1_ref, w2_ref, b2_ref,
                 wih0t_ref, bih0_ref, gi0_ref):
    x = jnp.maximum(jnp.dot(state_ref[...], w1_ref[...],
                            preferred_element_type=jnp.float32) + b1_ref[...], 0.0)
    m = jnp.maximum(jnp.dot(msg_ref[...], w2_ref[...],
                            preferred_element_type=jnp.float32) + b2_ref[...], 0.0)
    x = x + m
    gi0_ref[...] = jnp.dot(x, wih0t_ref[...],
                           preferred_element_type=jnp.float32) + bih0_ref[...]


def _encode(state, message, W1, b1, W2, b2, Wih0T, bih0):
    TB = 2000
    grid = (N // TB,)
    return pl.pallas_call(
        _encode_body,
        grid=grid,
        in_specs=[
            pl.BlockSpec((TB, D_STATE), lambda i: (i, 0)),
            pl.BlockSpec((TB, D_MSG), lambda i: (i, 0)),
            pl.BlockSpec((D_STATE, H), lambda i: (0, 0)),
            pl.BlockSpec((1, H), lambda i: (0, 0)),
            pl.BlockSpec((D_MSG, H), lambda i: (0, 0)),
            pl.BlockSpec((1, H), lambda i: (0, 0)),
            pl.BlockSpec((H, G3), lambda i: (0, 0)),
            pl.BlockSpec((1, G3), lambda i: (0, 0)),
        ],
        out_specs=pl.BlockSpec((TB, G3), lambda i: (i, 0)),
        out_shape=jax.ShapeDtypeStruct((N, G3), jnp.float32),
        interpret=_INTERPRET,
    )(state, message, W1, b1.reshape(1, H), W2, b2.reshape(1, H),
      Wih0T, bih0.reshape(1, G3))


def _gru_gates(gi, gh, h):
    r = jax.nn.sigmoid(gi[:, 0:H] + gh[:, 0:H])
    z = jax.nn.sigmoid(gi[:, H:2 * H] + gh[:, H:2 * H])
    n = jnp.tanh(gi[:, 2 * H:] + r * gh[:, 2 * H:])
    return (1.0 - z) * n + z * h


def _gru_body(gi0_ref, w3_ref, b3_ref, y_ref):
    w3 = w3_ref[...]
    b3 = jnp.broadcast_to(b3_ref[...], (8, 3 * G3))
    zeros8 = jnp.zeros((8, H), jnp.float32)

    gi00 = jnp.broadcast_to(gi0_ref[pl.ds(0, 1), :], (8, G3))
    h0 = _gru_gates(gi00, b3[:, 0:G3], zeros8)

    def fused(h0, h1):
        u = jnp.concatenate([h0, h1], axis=1).astype(jnp.bfloat16)
        return jnp.dot(u, w3, preferred_element_type=jnp.float32) + b3

    def step(g, carry):
        h0, h1 = carry
        m = fused(h0, h1)
        gi0t = jnp.broadcast_to(gi0_ref[pl.ds(g, 1), :], (8, G3))
        h0n = _gru_gates(gi0t, m[:, 0:G3], h0)
        h1n = _gru_gates(m[:, 2 * G3:], m[:, G3:2 * G3], h1)
        y_ref[pl.ds(g - 1, 1), :] = jnp.maximum(h1n[0:1, :], 0.0)
        return (h0n, h1n)

    h0, h1 = lax.fori_loop(1, N, step, (h0, zeros8))
    m = fused(h0, h1)
    h1n = _gru_gates(m[:, 2 * G3:], m[:, G3:2 * G3], h1)
    y_ref[pl.ds(N - 1, 1), :] = jnp.maximum(h1n[0:1, :], 0.0)


def _gru2(gi0, Whh0T, bhh0, Wih1T, bih1, Whh1T, bhh1):
    z = jnp.zeros((H, G3), jnp.float32)
    w3 = jnp.concatenate([
        jnp.concatenate([Whh0T, z, Wih1T], axis=1),
        jnp.concatenate([z, Whh1T, z], axis=1)], axis=0).astype(jnp.bfloat16)
    b3 = jnp.concatenate([bhh0, bhh1, bih1]).reshape(1, 3 * G3)
    return pl.pallas_call(
        _gru_body,
        grid=(1,),
        in_specs=[
            pl.BlockSpec((N, G3), lambda i: (0, 0)),
            pl.BlockSpec((2 * H, 3 * G3), lambda i: (0, 0)),
            pl.BlockSpec((1, 3 * G3), lambda i: (0, 0)),
        ],
        out_specs=pl.BlockSpec((N, H), lambda i: (0, 0)),
        out_shape=jax.ShapeDtypeStruct((N, H), jnp.float32),
        interpret=_INTERPRET,
    )(gi0, w3, b3)


def _gat_edge_body(hext_hbm, asv_hbm, adv_hbm, src_hbm, dst_hbm, zeros_hbm,
                   out_hbm, asv_t, adv_t, src_blk, dst_blk, rows, p_buf,
                   acc, gsem):
    c = lax.axis_index("c")
    s = lax.axis_index("s")
    wid = c * NS + s

    pltpu.sync_copy(asv_hbm, asv_t)
    pltpu.sync_copy(adv_hbm, adv_t)
    off = s * ROWS_PT
    pltpu.sync_copy(zeros_hbm.at[pl.ds(off, ROWS_PT)], acc.at[pl.ds(off, ROWS_PT)])

    @pl.when(s == NS - 1)
    def _zrem():
        pltpu.sync_copy(zeros_hbm.at[pl.ds(NS * ROWS_PT, ROWS_REM)],
                        acc.at[pl.ds(NS * ROWS_PT, ROWS_REM)])

    plsc.subcore_barrier()

    def block(b, _):
        base = wid * EPT + b * EB
        pltpu.sync_copy(src_hbm.at[pl.ds(base, EB)], src_blk)
        pltpu.sync_copy(dst_hbm.at[pl.ds(base, EB)], dst_blk)
        pltpu.async_copy(hext_hbm.at[src_blk], rows, gsem).wait()
        for g in range(EB // L):
            sidx = src_blk[pl.ds(g * L, L)]
            didx = dst_blk[pl.ds(g * L, L)]
            e = plsc.load_gather(asv_t, [sidx]) + plsc.load_gather(adv_t, [didx])
            e = jnp.where(e >= 0.0, e, 0.2 * e)
            p_buf[pl.ds(g * L, L)] = jnp.exp(e)

        def scale(i, _):
            pb = plsc.load_gather(p_buf, [jnp.zeros((L,), jnp.int32) + i])
            for j in range(DE // L):
                sl = pl.ds(j * L, L)
                rows[i, sl] = rows[i, sl] * pb
            return 0

        lax.fori_loop(0, EB, scale, 0)
        pltpu.sync_copy(rows, acc.at[dst_blk], add=True)
        return 0

    lax.fori_loop(0, NB, block, 0)
    plsc.subcore_barrier()
    pltpu.sync_copy(acc.at[pl.ds(off, ROWS_PT)], out_hbm.at[c, pl.ds(off, ROWS_PT)])

    @pl.when(s == NS - 1)
    def _orem():
        pltpu.sync_copy(acc.at[pl.ds(NS * ROWS_PT, ROWS_REM)],
                        out_hbm.at[c, pl.ds(NS * ROWS_PT, ROWS_REM)])


def _gat_edge(hext, asv, adv, src, dst, zeros):
    mesh = plsc.VectorSubcoreMesh(core_axis_name="c", subcore_axis_name="s",
                                  num_cores=NC, num_subcores=NS)
    return pl.kernel(
        _gat_edge_body,
        out_type=jax.ShapeDtypeStruct((NC, N, DE), jnp.float32),
        mesh=mesh,
        scratch_types=[
            pltpu.VMEM((N,), jnp.float32),
            pltpu.VMEM((N,), jnp.float32),
            pltpu.VMEM((EB,), jnp.int32),
            pltpu.VMEM((EB,), jnp.int32),
            pltpu.VMEM((EB, DE), jnp.float32),
            pltpu.VMEM((EB,), jnp.float32),
            pltpu.VMEM_SHARED((N, DE), jnp.float32),
            pltpu.SemaphoreType.DMA,
        ],
        compiler_params=pltpu.CompilerParams(needs_layout_passes=False,
                                             use_tc_tiling_on_sc=False),
        interpret=_INTERPRET,
    )(hext, asv, adv, src, dst, zeros)


def _prep_body(x_ref, w_ref, as_ref, ad_ref, hext_ref, asv_ref, adv_ref):
    h = jnp.dot(x_ref[...], w_ref[...], preferred_element_type=jnp.float32)
    asv_ref[...] = jnp.dot(h, as_ref[...], preferred_element_type=jnp.float32)
    adv_ref[...] = jnp.dot(h, ad_ref[...], preferred_element_type=jnp.float32)
    tb = h.shape[0]
    lane = lax.broadcasted_iota(jnp.int32, (tb, DE - H), 1)
    pad = jnp.where(lane == 0, 1.0, 0.0)
    hext_ref[...] = jnp.concatenate([h, pad], axis=1)


def _gat_prep(x, W, a_s, a_d):
    TB = 2000
    grid = (N // TB,)
    return pl.pallas_call(
        _prep_body,
        grid=grid,
        in_specs=[
            pl.BlockSpec((TB, H), lambda i: (i, 0)),
            pl.BlockSpec((H, H), lambda i: (0, 0)),
            pl.BlockSpec((H, 1), lambda i: (0, 0)),
            pl.BlockSpec((H, 1), lambda i: (0, 0)),
        ],
        out_specs=[
            pl.BlockSpec((TB, DE), lambda i: (i, 0)),
            pl.BlockSpec((TB, 1), lambda i: (i, 0)),
            pl.BlockSpec((TB, 1), lambda i: (i, 0)),
        ],
        out_shape=[
            jax.ShapeDtypeStruct((N, DE), jnp.float32),
            jax.ShapeDtypeStruct((N, 1), jnp.float32),
            jax.ShapeDtypeStruct((N, 1), jnp.float32),
        ],
        interpret=_INTERPRET,
    )(x, W, a_s.reshape(H, 1), a_d.reshape(H, 1))


def _mid_body(g0_ref, g1_ref, w_ref, as_ref, ad_ref,
              hext_ref, asv_ref, adv_ref):
    g = g0_ref[...] + g1_ref[...]
    den = g[:, H:H + 1]
    x1 = jnp.maximum(g[:, 0:H] / (den + 1e-16), 0.0)
    h = jnp.dot(x1, w_ref[...], preferred_element_type=jnp.float32)
    asv_ref[...] = jnp.dot(h, as_ref[...], preferred_element_type=jnp.float32)
    adv_ref[...] = jnp.dot(h, ad_ref[...], preferred_element_type=jnp.float32)
    tb = h.shape[0]
    lane = lax.broadcasted_iota(jnp.int32, (tb, DE - H), 1)
    pad = jnp.where(lane == 0, 1.0, 0.0)
    hext_ref[...] = jnp.concatenate([h, pad], axis=1)


def _gat_mid(g0, g1, W, a_s, a_d):
    TB = 2000
    grid = (N // TB,)
    return pl.pallas_call(
        _mid_body,
        grid=grid,
        in_specs=[
            pl.BlockSpec((TB, DE), lambda i: (i, 0)),
            pl.BlockSpec((TB, DE), lambda i: (i, 0)),
            pl.BlockSpec((H, H), lambda i: (0, 0)),
            pl.BlockSpec((H, 1), lambda i: (0, 0)),
            pl.BlockSpec((H, 1), lambda i: (0, 0)),
        ],
        out_specs=[
            pl.BlockSpec((TB, DE), lambda i: (i, 0)),
            pl.BlockSpec((TB, 1), lambda i: (i, 0)),
            pl.BlockSpec((TB, 1), lambda i: (i, 0)),
        ],
        out_shape=[
            jax.ShapeDtypeStruct((N, DE), jnp.float32),
            jax.ShapeDtypeStruct((N, 1), jnp.float32),
            jax.ShapeDtypeStruct((N, 1), jnp.float32),
        ],
        interpret=_INTERPRET,
    )(g0, g1, W, a_s.reshape(H, 1), a_d.reshape(H, 1))


def _final_body(g0_ref, g1_ref, y_ref, wc_ref, bc_ref, wmuy_ref, wmug_ref,
                bmu_ref, wmsg_ref, bmsg_ref, comm_ref, msg_ref, mu_ref):
    g = g0_ref[...] + g1_ref[...]
    den = g[:, H:H + 1]
    xg = g[:, 0:H] / (den + 1e-16)
    y = y_ref[...]
    comm_ref[...] = jax.nn.sigmoid(
        jnp.dot(xg, wc_ref[...], preferred_element_type=jnp.float32) + bc_ref[...])
    mu_ref[...] = jnp.tanh(
        jnp.dot(y, wmuy_ref[...], preferred_element_type=jnp.float32)
        + jnp.dot(xg, wmug_ref[...], preferred_element_type=jnp.float32)
        + bmu_ref[...])
    msg_ref[...] = jnp.tanh(
        jnp.dot(xg, wmsg_ref[...], preferred_element_type=jnp.float32) + bmsg_ref[...])


def _finalize(g0, g1, y, Wc, bc, Wmu, bmu, Wmsg, bmsg):
    TB = 2000
    grid = (N // TB,)
    NA = Wmu.shape[1]
    MS = Wmsg.shape[1]
    return pl.pallas_call(
        _final_body,
        grid=grid,
        in_specs=[
            pl.BlockSpec((TB, DE), lambda i: (i, 0)),
            pl.BlockSpec((TB, DE), lambda i: (i, 0)),
            pl.BlockSpec((TB, H), lambda i: (i, 0)),
            pl.BlockSpec((H, 1), lambda i: (0, 0)),
            pl.BlockSpec((1, 1), lambda i: (0, 0)),
            pl.BlockSpec((H, NA), lambda i: (0, 0)),
            pl.BlockSpec((H, NA), lambda i: (0, 0)),
            pl.BlockSpec((1, NA), lambda i: (0, 0)),
            pl.BlockSpec((H, MS), lambda i: (0, 0)),
            pl.BlockSpec((1, MS), lambda i: (0, 0)),
        ],
        out_specs=[
            pl.BlockSpec((TB, 1), lambda i: (i, 0)),
            pl.BlockSpec((TB, MS), lambda i: (i, 0)),
            pl.BlockSpec((TB, NA), lambda i: (i, 0)),
        ],
        out_shape=[
            jax.ShapeDtypeStruct((N, 1), jnp.float32),
            jax.ShapeDtypeStruct((N, MS), jnp.float32),
            jax.ShapeDtypeStruct((N, NA), jnp.float32),
        ],
        interpret=_INTERPRET,
    )(g0, g1, y, Wc, bc.reshape(1, 1), Wmu[:H], Wmu[H:],
      bmu.reshape(1, NA), Wmsg, bmsg.reshape(1, MS))


def kernel(state, message, edge_index, W1, b1, W2, b2, Wih0, Whh0, bih0, bhh0,
           Wih1, Whh1, bih1, bhh1, Wg1, a1s, a1d, Wg2, a2s, a2d, Wc, bc,
           Wmu, bmu, Wmsg, bmsg):
    gi0 = _encode(state, message, W1, b1, W2, b2, Wih0.T, bih0)
    y = _gru2(gi0, Whh0.T, bhh0, Wih1.T, bih1, Whh1.T, bhh1)

    src = edge_index[0]
    dst = edge_index[1]
    zeros = jnp.zeros((N, DE), jnp.float32)

    hext1, asv1, adv1 = _gat_prep(y, Wg1, a1s, a1d)
    g1 = _gat_edge(hext1, asv1.reshape(N), adv1.reshape(N), src, dst, zeros)
    hext2, asv2, adv2 = _gat_mid(g1[0], g1[1], Wg2, a2s, a2d)
    g2 = _gat_edge(hext2, asv2.reshape(N), adv2.reshape(N), src, dst, zeros)

    comm, msg_out, mu = _finalize(g2[0], g2[1], y, Wc, bc, Wmu, bmu, Wmsg, bmsg)
    return (comm, msg_out, mu)


# speedup vs baseline: 1.1349x; 1.1349x over previous
import functools

import jax
import jax.numpy as jnp
from jax import lax
from jax.experimental import pallas as pl
from jax.experimental.pallas import tpu as pltpu
from jax.experimental.pallas import tpu_sc as plsc

N = 10000
E = 320000
D_STATE = 128
D_MSG = 64
H = 128
G3 = 3 * H
DE = 144
NC, NS, L = 2, 16, 16
NW = NC * NS
EPT = E // NW
EB = 80
NB = EPT // EB
ROWS_PT = 624
ROWS_REM = N - NS * ROWS_PT

_INTERPRET = False


def _encode_body(state_ref, msg_ref, w1_ref, b1_ref, w2_ref, b2_ref,
                 wih0t_ref, bih0_ref, gi0_ref):
    x = jnp.maximum(jnp.dot(state_ref[...], w1_ref[...],
                            preferred_element_type=jnp.float32) + b1_ref[...], 0.0)
    m = jnp.maximum(jnp.dot(msg_ref[...], w2_ref[...],
                            preferred_element_type=jnp.float32) + b2_ref[...], 0.0)
    x = x + m
    gi0_ref[...] = jnp.dot(x, wih0t_ref[...],
                           preferred_element_type=jnp.float32) + bih0_ref[...]


def _encode(state, message, W1, b1, W2, b2, Wih0T, bih0):
    TB = 2000
    grid = (N // TB,)
    return pl.pallas_call(
        _encode_body,
        grid=grid,
        in_specs=[
            pl.BlockSpec((TB, D_STATE), lambda i: (i, 0)),
            pl.BlockSpec((TB, D_MSG), lambda i: (i, 0)),
            pl.BlockSpec((D_STATE, H), lambda i: (0, 0)),
            pl.BlockSpec((1, H), lambda i: (0, 0)),
            pl.BlockSpec((D_MSG, H), lambda i: (0, 0)),
            pl.BlockSpec((1, H), lambda i: (0, 0)),
            pl.BlockSpec((H, G3), lambda i: (0, 0)),
            pl.BlockSpec((1, G3), lambda i: (0, 0)),
        ],
        out_specs=pl.BlockSpec((TB, G3), lambda i: (i, 0)),
        out_shape=jax.ShapeDtypeStruct((N, G3), jnp.float32),
        interpret=_INTERPRET,
    )(state, message, W1, b1.reshape(1, H), W2, b2.reshape(1, H),
      Wih0T, bih0.reshape(1, G3))


def _gru_gates(gi, gh, h):
    r = jax.nn.sigmoid(gi[:, 0:H] + gh[:, 0:H])
    z = jax.nn.sigmoid(gi[:, H:2 * H] + gh[:, H:2 * H])
    n = jnp.tanh(gi[:, 2 * H:] + r * gh[:, 2 * H:])
    return (1.0 - z) * n + z * h


def _gru_body(gi0_ref, w3_ref, b3_ref, y_ref):
    w3 = w3_ref[...]
    b3 = jnp.broadcast_to(b3_ref[...], (8, 3 * G3))
    zeros8 = jnp.zeros((8, H), jnp.float32)

    gi00 = jnp.broadcast_to(gi0_ref[pl.ds(0, 1), :], (8, G3))
    h0 = _gru_gates(gi00, b3[:, 0:G3], zeros8)

    def fused(h0, h1):
        u = jnp.concatenate([h0, h1], axis=1).astype(jnp.bfloat16)
        return jnp.dot(u, w3, preferred_element_type=jnp.float32) + b3

    def step(g, carry):
        h0, h1 = carry
        m = fused(h0, h1)
        gi0t = jnp.broadcast_to(gi0_ref[pl.ds(g, 1), :], (8, G3))
        h0n = _gru_gates(gi0t, m[:, 0:G3], h0)
        h1n = _gru_gates(m[:, 2 * G3:], m[:, G3:2 * G3], h1)
        y_ref[pl.ds(g - 1, 1), :] = jnp.maximum(h1n[0:1, :], 0.0)
        return (h0n, h1n)

    h0, h1 = lax.fori_loop(1, N, step, (h0, zeros8))
    m = fused(h0, h1)
    h1n = _gru_gates(m[:, 2 * G3:], m[:, G3:2 * G3], h1)
    y_ref[pl.ds(N - 1, 1), :] = jnp.maximum(h1n[0:1, :], 0.0)


def _gru2(gi0, Whh0T, bhh0, Wih1T, bih1, Whh1T, bhh1):
    z = jnp.zeros((H, G3), jnp.float32)
    w3 = jnp.concatenate([
        jnp.concatenate([Whh0T, z, Wih1T], axis=1),
        jnp.concatenate([z, Whh1T, z], axis=1)], axis=0).astype(jnp.bfloat16)
    b3 = jnp.concatenate([bhh0, bhh1, bih1]).reshape(1, 3 * G3)
    return pl.pallas_call(
        _gru_body,
        grid=(1,),
        in_specs=[
            pl.BlockSpec((N, G3), lambda i: (0, 0)),
            pl.BlockSpec((2 * H, 3 * G3), lambda i: (0, 0)),
            pl.BlockSpec((1, 3 * G3), lambda i: (0, 0)),
        ],
        out_specs=pl.BlockSpec((N, H), lambda i: (0, 0)),
        out_shape=jax.ShapeDtypeStruct((N, H), jnp.float32),
        interpret=_INTERPRET,
    )(gi0, w3, b3)


def _gat_edge_body(hext_hbm, adv_hbm, src_hbm, dst_hbm, zeros_hbm,
                   out_hbm, adv_t, src_a, dst_a, src_b, dst_b,
                   rows_a, rows_b, p_buf, acc, gsem_a, gsem_b, ssem_a, ssem_b):
    c = lax.axis_index("c")
    s = lax.axis_index("s")
    wid = c * NS + s

    pltpu.sync_copy(adv_hbm, adv_t)
    off = s * ROWS_PT
    pltpu.sync_copy(zeros_hbm.at[pl.ds(off, ROWS_PT)], acc.at[pl.ds(off, ROWS_PT)])

    @pl.when(s == NS - 1)
    def _zrem():
        pltpu.sync_copy(zeros_hbm.at[pl.ds(NS * ROWS_PT, ROWS_REM)],
                        acc.at[pl.ds(NS * ROWS_PT, ROWS_REM)])

    plsc.subcore_barrier()

    def load_idx(k, sbuf, dbuf):
        base = wid * EPT + k * EB
        pltpu.sync_copy(src_hbm.at[pl.ds(base, EB)], sbuf)
        pltpu.sync_copy(dst_hbm.at[pl.ds(base, EB)], dbuf)

    def start_gather(sbuf, rows, gsem):
        pltpu.async_copy(hext_hbm.at[sbuf], rows, gsem)

    def wait_gather(sbuf, rows, gsem):
        pltpu.make_async_copy(hext_hbm.at[sbuf], rows, gsem).wait()

    def start_scatter(rows, dbuf, ssem):
        pltpu.async_copy(rows, acc.at[dbuf], ssem, add=True)

    def wait_scatter(rows, dbuf, ssem):
        pltpu.make_async_copy(rows, acc.at[dbuf], ssem).wait()

    def compute(sbuf, dbuf, rows):
        lane = lax.iota(jnp.int32, L)
        col_as = jnp.zeros((L,), jnp.int32) + (H + 1)
        for g in range(EB // L):
            didx = dbuf[pl.ds(g * L, L)]
            sv = plsc.load_gather(rows, [lane + g * L, col_as])
            e = sv + plsc.load_gather(adv_t, [didx])
            e = jnp.where(e >= 0.0, e, 0.2 * e)
            p_buf[pl.ds(g * L, L)] = jnp.exp(e)

        def scale(i, _):
            pb = plsc.load_gather(p_buf, [jnp.zeros((L,), jnp.int32) + i])
            for j in range(DE // L):
                sl = pl.ds(j * L, L)
                rows[i, sl] = rows[i, sl] * pb
            return 0

        lax.fori_loop(0, EB, scale, 0)

    load_idx(0, src_a, dst_a)
    start_gather(src_a, rows_a, gsem_a)
    load_idx(1, src_b, dst_b)
    start_gather(src_b, rows_b, gsem_b)

    def pair(j, _):
        ka = 2 * j
        wait_gather(src_a, rows_a, gsem_a)
        compute(src_a, dst_a, rows_a)
        start_scatter(rows_a, dst_a, ssem_a)

        wait_gather(src_b, rows_b, gsem_b)
        compute(src_b, dst_b, rows_b)
        start_scatter(rows_b, dst_b, ssem_b)

        wait_scatter(rows_a, dst_a, ssem_a)
        load_idx(ka + 2, src_a, dst_a)
        start_gather(src_a, rows_a, gsem_a)

        @pl.when(j < (NB - 3) // 2)
        def _prep_b():
            wait_scatter(rows_b, dst_b, ssem_b)
            load_idx(ka + 3, src_b, dst_b)
            start_gather(src_b, rows_b, gsem_b)

        return 0

    lax.fori_loop(0, (NB - 1) // 2, pair, 0)
    wait_gather(src_a, rows_a, gsem_a)
    compute(src_a, dst_a, rows_a)
    start_scatter(rows_a, dst_a, ssem_a)
    wait_scatter(rows_a, dst_a, ssem_a)
    wait_scatter(rows_b, dst_b, ssem_b)
    plsc.subcore_barrier()
    pltpu.sync_copy(acc.at[pl.ds(off, ROWS_PT)], out_hbm.at[c, pl.ds(off, ROWS_PT)])

    @pl.when(s == NS - 1)
    def _orem():
        pltpu.sync_copy(acc.at[pl.ds(NS * ROWS_PT, ROWS_REM)],
                        out_hbm.at[c, pl.ds(NS * ROWS_PT, ROWS_REM)])


def _gat_edge(hext, adv, src, dst, zeros):
    mesh = plsc.VectorSubcoreMesh(core_axis_name="c", subcore_axis_name="s",
                                  num_cores=NC, num_subcores=NS)
    return pl.kernel(
        _gat_edge_body,
        out_type=jax.ShapeDtypeStruct((NC, N, DE), jnp.float32),
        mesh=mesh,
        scratch_types=[
            pltpu.VMEM((N,), jnp.float32),
            pltpu.VMEM((EB,), jnp.int32),
            pltpu.VMEM((EB,), jnp.int32),
            pltpu.VMEM((EB,), jnp.int32),
            pltpu.VMEM((EB,), jnp.int32),
            pltpu.VMEM((EB, DE), jnp.float32),
            pltpu.VMEM((EB, DE), jnp.float32),
            pltpu.VMEM((EB,), jnp.float32),
            pltpu.VMEM_SHARED((N, DE), jnp.float32),
            pltpu.SemaphoreType.DMA,
            pltpu.SemaphoreType.DMA,
            pltpu.SemaphoreType.DMA,
            pltpu.SemaphoreType.DMA,
        ],
        compiler_params=pltpu.CompilerParams(needs_layout_passes=False,
                                             use_tc_tiling_on_sc=False),
        interpret=_INTERPRET,
    )(hext, adv, src, dst, zeros)


def _prep_body(x_ref, w_ref, as_ref, ad_ref, hext_ref, adv_ref):
    h = jnp.dot(x_ref[...], w_ref[...], preferred_element_type=jnp.float32)
    asv = jnp.dot(h, as_ref[...], preferred_element_type=jnp.float32)
    adv_ref[...] = jnp.dot(h, ad_ref[...], preferred_element_type=jnp.float32)
    tb = h.shape[0]
    lane = lax.broadcasted_iota(jnp.int32, (tb, DE - H), 1)
    pad = jnp.where(lane == 0, 1.0, jnp.where(lane == 1, asv, 0.0))
    hext_ref[...] = jnp.concatenate([h, pad], axis=1)


def _gat_prep(x, W, a_s, a_d):
    TB = 2000
    grid = (N // TB,)
    return pl.pallas_call(
        _prep_body,
        grid=grid,
        in_specs=[
            pl.BlockSpec((TB, H), lambda i: (i, 0)),
            pl.BlockSpec((H, H), lambda i: (0, 0)),
            pl.BlockSpec((H, 1), lambda i: (0, 0)),
            pl.BlockSpec((H, 1), lambda i: (0, 0)),
        ],
        out_specs=[
            pl.BlockSpec((TB, DE), lambda i: (i, 0)),
            pl.BlockSpec((TB, 1), lambda i: (i, 0)),
        ],
        out_shape=[
            jax.ShapeDtypeStruct((N, DE), jnp.float32),
            jax.ShapeDtypeStruct((N, 1), jnp.float32),
        ],
        interpret=_INTERPRET,
    )(x, W, a_s.reshape(H, 1), a_d.reshape(H, 1))


def _mid_body(g0_ref, g1_ref, w_ref, as_ref, ad_ref,
              hext_ref, adv_ref):
    g = g0_ref[0] + g1_ref[0]
    den = g[:, H:H + 1]
    x1 = jnp.maximum(g[:, 0:H] / (den + 1e-16), 0.0)
    h = jnp.dot(x1, w_ref[...], preferred_element_type=jnp.float32)
    asv = jnp.dot(h, as_ref[...], preferred_element_type=jnp.float32)
    adv_ref[...] = jnp.dot(h, ad_ref[...], preferred_element_type=jnp.float32)
    tb = h.shape[0]
    lane = lax.broadcasted_iota(jnp.int32, (tb, DE - H), 1)
    pad = jnp.where(lane == 0, 1.0, jnp.where(lane == 1, asv, 0.0))
    hext_ref[...] = jnp.concatenate([h, pad], axis=1)


def _gat_mid(g0, g1, W, a_s, a_d):
    TB = 2000
    grid = (N // TB,)
    return pl.pallas_call(
        _mid_body,
        grid=grid,
        in_specs=[
            pl.BlockSpec((1, TB, DE), lambda i: (0, i, 0)),
            pl.BlockSpec((1, TB, DE), lambda i: (1, i, 0)),
            pl.BlockSpec((H, H), lambda i: (0, 0)),
            pl.BlockSpec((H, 1), lambda i: (0, 0)),
            pl.BlockSpec((H, 1), lambda i: (0, 0)),
        ],
        out_specs=[
            pl.BlockSpec((TB, DE), lambda i: (i, 0)),
            pl.BlockSpec((TB, 1), lambda i: (i, 0)),
        ],
        out_shape=[
            jax.ShapeDtypeStruct((N, DE), jnp.float32),
            jax.ShapeDtypeStruct((N, 1), jnp.float32),
        ],
        interpret=_INTERPRET,
    )(g0, g1, W, a_s.reshape(H, 1), a_d.reshape(H, 1))


def _final_body(g0_ref, g1_ref, y_ref, wc_ref, bc_ref, wmuy_ref, wmug_ref,
                bmu_ref, wmsg_ref, bmsg_ref, comm_ref, msg_ref, mu_ref):
    g = g0_ref[0] + g1_ref[0]
    den = g[:, H:H + 1]
    xg = g[:, 0:H] / (den + 1e-16)
    y = y_ref[...]
    comm_ref[...] = jax.nn.sigmoid(
        jnp.dot(xg, wc_ref[...], preferred_element_type=jnp.float32) + bc_ref[...])
    mu_ref[...] = jnp.tanh(
        jnp.dot(y, wmuy_ref[...], preferred_element_type=jnp.float32)
        + jnp.dot(xg, wmug_ref[...], preferred_element_type=jnp.float32)
        + bmu_ref[...])
    msg_ref[...] = jnp.tanh(
        jnp.dot(xg, wmsg_ref[...], preferred_element_type=jnp.float32) + bmsg_ref[...])


def _finalize(g0, g1, y, Wc, bc, Wmu, bmu, Wmsg, bmsg):
    TB = 2000
    grid = (N // TB,)
    NA = Wmu.shape[1]
    MS = Wmsg.shape[1]
    return pl.pallas_call(
        _final_body,
        grid=grid,
        in_specs=[
            pl.BlockSpec((1, TB, DE), lambda i: (0, i, 0)),
            pl.BlockSpec((1, TB, DE), lambda i: (1, i, 0)),
            pl.BlockSpec((TB, H), lambda i: (i, 0)),
            pl.BlockSpec((H, 1), lambda i: (0, 0)),
            pl.BlockSpec((1, 1), lambda i: (0, 0)),
            pl.BlockSpec((H, NA), lambda i: (0, 0)),
            pl.BlockSpec((H, NA), lambda i: (0, 0)),
            pl.BlockSpec((1, NA), lambda i: (0, 0)),
            pl.BlockSpec((H, MS), lambda i: (0, 0)),
            pl.BlockSpec((1, MS), lambda i: (0, 0)),
        ],
        out_specs=[
            pl.BlockSpec((TB, 1), lambda i: (i, 0)),
            pl.BlockSpec((TB, MS), lambda i: (i, 0)),
            pl.BlockSpec((TB, NA), lambda i: (i, 0)),
        ],
        out_shape=[
            jax.ShapeDtypeStruct((N, 1), jnp.float32),
            jax.ShapeDtypeStruct((N, MS), jnp.float32),
            jax.ShapeDtypeStruct((N, NA), jnp.float32),
        ],
        interpret=_INTERPRET,
    )(g0, g1, y, Wc, bc.reshape(1, 1), Wmu[:H], Wmu[H:],
      bmu.reshape(1, NA), Wmsg, bmsg.reshape(1, MS))


def kernel(state, message, edge_index, W1, b1, W2, b2, Wih0, Whh0, bih0, bhh0,
           Wih1, Whh1, bih1, bhh1, Wg1, a1s, a1d, Wg2, a2s, a2d, Wc, bc,
           Wmu, bmu, Wmsg, bmsg):
    gi0 = _encode(state, message, W1, b1, W2, b2, Wih0.T, bih0)
    y = _gru2(gi0, Whh0.T, bhh0, Wih1.T, bih1, Whh1.T, bhh1)

    src = edge_index[0]
    dst = edge_index[1]
    zeros = jnp.zeros((N, DE), jnp.float32)

    hext1, adv1 = _gat_prep(y, Wg1, a1s, a1d)
    g1 = _gat_edge(hext1, adv1.reshape(N), src, dst, zeros)
    hext2, adv2 = _gat_mid(g1, g1, Wg2, a2s, a2d)
    g2 = _gat_edge(hext2, adv2.reshape(N), src, dst, zeros)

    comm, msg_out, mu = _finalize(g2, g2, y, Wc, bc, Wmu, bmu, Wmsg, bmsg)
    return (comm, msg_out, mu)


# speedup vs baseline: 1.1354x; 1.0004x over previous
import jax
import jax.numpy as jnp
from jax import lax
from jax.experimental import pallas as pl
from jax.experimental.pallas import tpu as pltpu
from jax.experimental.pallas import tpu_sc as plsc

N = 10000
E = 320000
D_STATE = 128
D_MSG = 64
H = 128
G3 = 3 * H
DE = 144
NC, NS, L = 2, 16, 16
NW = NC * NS
EPT = E // NW
EB = 80
NB = EPT // EB
ROWS_PT = 624
ROWS_REM = N - NS * ROWS_PT


def _encode_body(state_ref, msg_ref, w1_ref, b1_ref, w2_ref, b2_ref,
                 wih0t_ref, bih0_ref, gi0_ref):
    x = jnp.maximum(jnp.dot(state_ref[...], w1_ref[...],
                            preferred_element_type=jnp.float32) + b1_ref[...], 0.0)
    m = jnp.maximum(jnp.dot(msg_ref[...], w2_ref[...],
                            preferred_element_type=jnp.float32) + b2_ref[...], 0.0)
    x = x + m
    gi0_ref[...] = jnp.dot(x, wih0t_ref[...],
                           preferred_element_type=jnp.float32) + bih0_ref[...]


def _encode(state, message, W1, b1, W2, b2, Wih0T, bih0):
    TB = 2000
    grid = (N // TB,)
    return pl.pallas_call(
        _encode_body,
        grid=grid,
        in_specs=[
            pl.BlockSpec((TB, D_STATE), lambda i: (i, 0)),
            pl.BlockSpec((TB, D_MSG), lambda i: (i, 0)),
            pl.BlockSpec((D_STATE, H), lambda i: (0, 0)),
            pl.BlockSpec((1, H), lambda i: (0, 0)),
            pl.BlockSpec((D_MSG, H), lambda i: (0, 0)),
            pl.BlockSpec((1, H), lambda i: (0, 0)),
            pl.BlockSpec((H, G3), lambda i: (0, 0)),
            pl.BlockSpec((1, G3), lambda i: (0, 0)),
        ],
        out_specs=pl.BlockSpec((TB, G3), lambda i: (i, 0)),
        out_shape=jax.ShapeDtypeStruct((N, G3), jnp.float32),
    )(state, message, W1, b1.reshape(1, H), W2, b2.reshape(1, H),
      Wih0T, bih0.reshape(1, G3))


def _gru_gates(gi, gh, h):
    r = jax.nn.sigmoid(gi[:, 0:H] + gh[:, 0:H])
    z = jax.nn.sigmoid(gi[:, H:2 * H] + gh[:, H:2 * H])
    n = jnp.tanh(gi[:, 2 * H:] + r * gh[:, 2 * H:])
    return (1.0 - z) * n + z * h


def _gru_body(gi0_ref, w3_ref, b3_ref, y_ref):
    w3 = w3_ref[...]
    b3 = jnp.broadcast_to(b3_ref[...], (8, 3 * G3))
    zeros8 = jnp.zeros((8, H), jnp.float32)

    gi00 = jnp.broadcast_to(gi0_ref[pl.ds(0, 1), :], (8, G3))
    h0 = _gru_gates(gi00, b3[:, 0:G3], zeros8)

    def fused(h0, h1):
        u = jnp.concatenate([h0, h1], axis=1).astype(jnp.bfloat16)
        return jnp.dot(u, w3, preferred_element_type=jnp.float32) + b3

    def step(g, carry):
        h0, h1 = carry
        m = fused(h0, h1)
        gi0t = jnp.broadcast_to(gi0_ref[pl.ds(g, 1), :], (8, G3))
        h0n = _gru_gates(gi0t, m[:, 0:G3], h0)
        h1n = _gru_gates(m[:, 2 * G3:], m[:, G3:2 * G3], h1)
        y_ref[pl.ds(g - 1, 1), :] = jnp.maximum(h1n[0:1, :], 0.0)
        return (h0n, h1n)

    h0, h1 = lax.fori_loop(1, N, step, (h0, zeros8))
    m = fused(h0, h1)
    h1n = _gru_gates(m[:, 2 * G3:], m[:, G3:2 * G3], h1)
    y_ref[pl.ds(N - 1, 1), :] = jnp.maximum(h1n[0:1, :], 0.0)


def _gru2(gi0, Whh0T, bhh0, Wih1T, bih1, Whh1T, bhh1):
    z = jnp.zeros((H, G3), jnp.float32)
    w3 = jnp.concatenate([
        jnp.concatenate([Whh0T, z, Wih1T], axis=1),
        jnp.concatenate([z, Whh1T, z], axis=1)], axis=0).astype(jnp.bfloat16)
    b3 = jnp.concatenate([bhh0, bhh1, bih1]).reshape(1, 3 * G3)
    return pl.pallas_call(
        _gru_body,
        grid=(1,),
        in_specs=[
            pl.BlockSpec((N, G3), lambda i: (0, 0)),
            pl.BlockSpec((2 * H, 3 * G3), lambda i: (0, 0)),
            pl.BlockSpec((1, 3 * G3), lambda i: (0, 0)),
        ],
        out_specs=pl.BlockSpec((N, H), lambda i: (0, 0)),
        out_shape=jax.ShapeDtypeStruct((N, H), jnp.float32),
    )(gi0, w3, b3)


def _gat_edge_body(hext_hbm, adv_hbm, src_hbm, dst_hbm, zeros_hbm,
                   out_hbm, adv_t, src_a, dst_a, src_b, dst_b,
                   rows_a, rows_b, p_buf, acc, gsem_a, gsem_b, ssem_a, ssem_b):
    c = lax.axis_index("c")
    s = lax.axis_index("s")
    wid = c * NS + s

    pltpu.sync_copy(adv_hbm, adv_t)
    off = s * ROWS_PT
    pltpu.sync_copy(zeros_hbm.at[pl.ds(off, ROWS_PT)], acc.at[pl.ds(off, ROWS_PT)])

    @pl.when(s == NS - 1)
    def _zrem():
        pltpu.sync_copy(zeros_hbm.at[pl.ds(NS * ROWS_PT, ROWS_REM)],
                        acc.at[pl.ds(NS * ROWS_PT, ROWS_REM)])

    plsc.subcore_barrier()

    def load_idx(k, sbuf, dbuf):
        base = wid * EPT + k * EB
        pltpu.sync_copy(src_hbm.at[pl.ds(base, EB)], sbuf)
        pltpu.sync_copy(dst_hbm.at[pl.ds(base, EB)], dbuf)

    def start_gather(sbuf, rows, gsem):
        pltpu.async_copy(hext_hbm.at[sbuf], rows, gsem)

    def wait_gather(sbuf, rows, gsem):
        pltpu.make_async_copy(hext_hbm.at[sbuf], rows, gsem).wait()

    def start_scatter(rows, dbuf, ssem):
        pltpu.async_copy(rows, acc.at[dbuf], ssem, add=True)

    def wait_scatter(rows, dbuf, ssem):
        pltpu.make_async_copy(rows, acc.at[dbuf], ssem).wait()

    def compute(sbuf, dbuf, rows):
        lane = lax.iota(jnp.int32, L)
        col_as = jnp.zeros((L,), jnp.int32) + (H + 1)
        for g in range(EB // L):
            didx = dbuf[pl.ds(g * L, L)]
            sv = plsc.load_gather(rows, [lane + g * L, col_as])
            e = sv + plsc.load_gather(adv_t, [didx])
            e = jnp.where(e >= 0.0, e, 0.2 * e)
            p_buf[pl.ds(g * L, L)] = jnp.exp(e)

        def scale(i, _):
            pb = plsc.load_gather(p_buf, [jnp.zeros((L,), jnp.int32) + i])
            for j in range(DE // L):
                sl = pl.ds(j * L, L)
                rows[i, sl] = rows[i, sl] * pb
            return 0

        lax.fori_loop(0, EB, scale, 0)

    load_idx(0, src_a, dst_a)
    start_gather(src_a, rows_a, gsem_a)
    load_idx(1, src_b, dst_b)
    start_gather(src_b, rows_b, gsem_b)

    def pair(j, _):
        ka = 2 * j
        wait_gather(src_a, rows_a, gsem_a)
        compute(src_a, dst_a, rows_a)
        start_scatter(rows_a, dst_a, ssem_a)

        wait_gather(src_b, rows_b, gsem_b)
        compute(src_b, dst_b, rows_b)
        start_scatter(rows_b, dst_b, ssem_b)

        wait_scatter(rows_a, dst_a, ssem_a)
        load_idx(ka + 2, src_a, dst_a)
        start_gather(src_a, rows_a, gsem_a)

        @pl.when(j < (NB - 3) // 2)
        def _prep_b():
            wait_scatter(rows_b, dst_b, ssem_b)
            load_idx(ka + 3, src_b, dst_b)
            start_gather(src_b, rows_b, gsem_b)

        return 0

    lax.fori_loop(0, (NB - 1) // 2, pair, 0)
    wait_gather(src_a, rows_a, gsem_a)
    compute(src_a, dst_a, rows_a)
    start_scatter(rows_a, dst_a, ssem_a)
    wait_scatter(rows_a, dst_a, ssem_a)
    wait_scatter(rows_b, dst_b, ssem_b)
    plsc.subcore_barrier()
    pltpu.sync_copy(acc.at[pl.ds(off, ROWS_PT)], out_hbm.at[c, pl.ds(off, ROWS_PT)])

    @pl.when(s == NS - 1)
    def _orem():
        pltpu.sync_copy(acc.at[pl.ds(NS * ROWS_PT, ROWS_REM)],
                        out_hbm.at[c, pl.ds(NS * ROWS_PT, ROWS_REM)])


def _gat_edge(hext, adv, src, dst, zeros):
    mesh = plsc.VectorSubcoreMesh(core_axis_name="c", subcore_axis_name="s",
                                  num_cores=NC, num_subcores=NS)
    return pl.kernel(
        _gat_edge_body,
        out_type=jax.ShapeDtypeStruct((NC, N, DE), jnp.float32),
        mesh=mesh,
        scratch_types=[
            pltpu.VMEM((N,), jnp.float32),
            pltpu.VMEM((EB,), jnp.int32),
            pltpu.VMEM((EB,), jnp.int32),
            pltpu.VMEM((EB,), jnp.int32),
            pltpu.VMEM((EB,), jnp.int32),
            pltpu.VMEM((EB, DE), jnp.float32),
            pltpu.VMEM((EB, DE), jnp.float32),
            pltpu.VMEM((EB,), jnp.float32),
            pltpu.VMEM_SHARED((N, DE), jnp.float32),
            pltpu.SemaphoreType.DMA,
            pltpu.SemaphoreType.DMA,
            pltpu.SemaphoreType.DMA,
            pltpu.SemaphoreType.DMA,
        ],
        compiler_params=pltpu.CompilerParams(needs_layout_passes=False,
                                             use_tc_tiling_on_sc=False),
    )(hext, adv, src, dst, zeros)


def _prep_body(x_ref, w_ref, as_ref, ad_ref, hext_ref, adv_ref):
    h = jnp.dot(x_ref[...], w_ref[...], preferred_element_type=jnp.float32)
    asv = jnp.dot(h, as_ref[...], preferred_element_type=jnp.float32)
    adv_ref[...] = jnp.dot(h, ad_ref[...], preferred_element_type=jnp.float32)
    tb = h.shape[0]
    lane = lax.broadcasted_iota(jnp.int32, (tb, DE - H), 1)
    pad = jnp.where(lane == 0, 1.0, jnp.where(lane == 1, asv, 0.0))
    hext_ref[...] = jnp.concatenate([h, pad], axis=1)


def _gat_prep(x, W, a_s, a_d):
    TB = 2000
    grid = (N // TB,)
    return pl.pallas_call(
        _prep_body,
        grid=grid,
        in_specs=[
            pl.BlockSpec((TB, H), lambda i: (i, 0)),
            pl.BlockSpec((H, H), lambda i: (0, 0)),
            pl.BlockSpec((H, 1), lambda i: (0, 0)),
            pl.BlockSpec((H, 1), lambda i: (0, 0)),
        ],
        out_specs=[
            pl.BlockSpec((TB, DE), lambda i: (i, 0)),
            pl.BlockSpec((TB, 1), lambda i: (i, 0)),
        ],
        out_shape=[
            jax.ShapeDtypeStruct((N, DE), jnp.float32),
            jax.ShapeDtypeStruct((N, 1), jnp.float32),
        ],
    )(x, W, a_s.reshape(H, 1), a_d.reshape(H, 1))


def _mid_body(g0_ref, g1_ref, w_ref, as_ref, ad_ref,
              hext_ref, adv_ref):
    g = g0_ref[0] + g1_ref[0]
    den = g[:, H:H + 1]
    x1 = jnp.maximum(g[:, 0:H] / (den + 1e-16), 0.0)
    h = jnp.dot(x1, w_ref[...], preferred_element_type=jnp.float32)
    asv = jnp.dot(h, as_ref[...], preferred_element_type=jnp.float32)
    adv_ref[...] = jnp.dot(h, ad_ref[...], preferred_element_type=jnp.float32)
    tb = h.shape[0]
    lane = lax.broadcasted_iota(jnp.int32, (tb, DE - H), 1)
    pad = jnp.where(lane == 0, 1.0, jnp.where(lane == 1, asv, 0.0))
    hext_ref[...] = jnp.concatenate([h, pad], axis=1)


def _gat_mid(g0, g1, W, a_s, a_d):
    TB = 2000
    grid = (N // TB,)
    return pl.pallas_call(
        _mid_body,
        grid=grid,
        in_specs=[
            pl.BlockSpec((1, TB, DE), lambda i: (0, i, 0)),
            pl.BlockSpec((1, TB, DE), lambda i: (1, i, 0)),
            pl.BlockSpec((H, H), lambda i: (0, 0)),
            pl.BlockSpec((H, 1), lambda i: (0, 0)),
            pl.BlockSpec((H, 1), lambda i: (0, 0)),
        ],
        out_specs=[
            pl.BlockSpec((TB, DE), lambda i: (i, 0)),
            pl.BlockSpec((TB, 1), lambda i: (i, 0)),
        ],
        out_shape=[
            jax.ShapeDtypeStruct((N, DE), jnp.float32),
            jax.ShapeDtypeStruct((N, 1), jnp.float32),
        ],
    )(g0, g1, W, a_s.reshape(H, 1), a_d.reshape(H, 1))


def _final_body(g0_ref, g1_ref, y_ref, wc_ref, bc_ref, wmuy_ref, wmug_ref,
                bmu_ref, wmsg_ref, bmsg_ref, comm_ref, msg_ref, mu_ref):
    g = g0_ref[0] + g1_ref[0]
    den = g[:, H:H + 1]
    xg = g[:, 0:H] / (den + 1e-16)
    y = y_ref[...]
    comm_ref[...] = jax.nn.sigmoid(
        jnp.dot(xg, wc_ref[...], preferred_element_type=jnp.float32) + bc_ref[...])
    mu_ref[...] = jnp.tanh(
        jnp.dot(y, wmuy_ref[...], preferred_element_type=jnp.float32)
        + jnp.dot(xg, wmug_ref[...], preferred_element_type=jnp.float32)
        + bmu_ref[...])
    msg_ref[...] = jnp.tanh(
        jnp.dot(xg, wmsg_ref[...], preferred_element_type=jnp.float32) + bmsg_ref[...])


def _finalize(g0, g1, y, Wc, bc, Wmu, bmu, Wmsg, bmsg):
    TB = 2000
    grid = (N // TB,)
    NA = Wmu.shape[1]
    MS = Wmsg.shape[1]
    return pl.pallas_call(
        _final_body,
        grid=grid,
        in_specs=[
            pl.BlockSpec((1, TB, DE), lambda i: (0, i, 0)),
            pl.BlockSpec((1, TB, DE), lambda i: (1, i, 0)),
            pl.BlockSpec((TB, H), lambda i: (i, 0)),
            pl.BlockSpec((H, 1), lambda i: (0, 0)),
            pl.BlockSpec((1, 1), lambda i: (0, 0)),
            pl.BlockSpec((H, NA), lambda i: (0, 0)),
            pl.BlockSpec((H, NA), lambda i: (0, 0)),
            pl.BlockSpec((1, NA), lambda i: (0, 0)),
            pl.BlockSpec((H, MS), lambda i: (0, 0)),
            pl.BlockSpec((1, MS), lambda i: (0, 0)),
        ],
        out_specs=[
            pl.BlockSpec((TB, 1), lambda i: (i, 0)),
            pl.BlockSpec((TB, MS), lambda i: (i, 0)),
            pl.BlockSpec((TB, NA), lambda i: (i, 0)),
        ],
        out_shape=[
            jax.ShapeDtypeStruct((N, 1), jnp.float32),
            jax.ShapeDtypeStruct((N, MS), jnp.float32),
            jax.ShapeDtypeStruct((N, NA), jnp.float32),
        ],
    )(g0, g1, y, Wc, bc.reshape(1, 1), Wmu[:H], Wmu[H:],
      bmu.reshape(1, NA), Wmsg, bmsg.reshape(1, MS))


def kernel(state, message, edge_index, W1, b1, W2, b2, Wih0, Whh0, bih0, bhh0,
           Wih1, Whh1, bih1, bhh1, Wg1, a1s, a1d, Wg2, a2s, a2d, Wc, bc,
           Wmu, bmu, Wmsg, bmsg):
    gi0 = _encode(state, message, W1, b1, W2, b2, Wih0.T, bih0)
    y = _gru2(gi0, Whh0.T, bhh0, Wih1.T, bih1, Whh1.T, bhh1)

    src = edge_index[0]
    dst = edge_index[1]
    zeros = jnp.zeros((N, DE), jnp.float32)

    hext1, adv1 = _gat_prep(y, Wg1, a1s, a1d)
    g1 = _gat_edge(hext1, adv1.reshape(N), src, dst, zeros)
    hext2, adv2 = _gat_mid(g1, g1, Wg2, a2s, a2d)
    g2 = _gat_edge(hext2, adv2.reshape(N), src, dst, zeros)

    comm, msg_out, mu = _finalize(g2, g2, y, Wc, bc, Wmu, bmu, Wmsg, bmsg)
    return (comm, msg_out, mu)
